# baseline (speedup 1.0000x reference)
import functools

import jax
import jax.numpy as jnp
from jax import lax
from jax.experimental import pallas as pl
from jax.experimental.pallas import tpu as pltpu
from jax.experimental.pallas import tpu_sc as plsc

_NC = 2
_NS = 16
_NCH = 14
_L = 16
_NPV = 9


def _sc_body(co_x, co_y, x_hbm, y_hbm, params_h, ox_hbm, oy_hbm,
             xbuf, ybuf, oxbuf, oybuf, pbuf, sem):
    B, C, NCH, CHW = x_hbm.shape
    wid = lax.axis_index("s") * _NC + lax.axis_index("c")
    per = B // (_NC * _NS)

    pltpu.sync_copy(params_h, pbuf)
    pv = [pbuf[pl.ds(i * _L, _L)] for i in range(_NPV)]

    def scal(k):
        return pv[k // _L][k % _L]

    o_wr1, o_br1, o_wr2, o_br2 = 0, 8, 10, 14
    o_we1 = 16
    o_be1 = o_we1 + co_x * C
    o_we2 = o_be1 + co_x
    o_be2 = o_we2 + co_y * C
    o_we3 = o_be2 + co_y
    o_be3 = o_we3 + co_x * C
    o_we4 = o_be3 + co_x
    o_be4 = o_we4 + co_y * C

    inv = jnp.float32(1.0 / (NCH * CHW))
    nvec = CHW // _L
    zvec = jnp.zeros(_L, jnp.float32)

    for bi in range(per):
        b = wid * per + bi

        def accumulate(src, buf):
            def redk(k, accs):
                pltpu.async_copy(src.at[b, :, k, :], buf, sem).wait()

                def redc(c):
                    def redj(j, a):
                        return a + buf[c, pl.ds(j * _L, _L)]
                    return lax.fori_loop(0, nvec, redj, accs[c])

                return tuple(redc(c) for c in range(C))

            accs = lax.fori_loop(0, NCH, redk, (zvec,) * C)

            def lanesum(a):
                s = a[0]
                for i in range(1, _L):
                    s = s + a[i]
                return s

            return [lanesum(a) * inv for a in accs]

        mxs = accumulate(x_hbm, xbuf)
        mys = accumulate(y_hbm, ybuf)

        def gates(m):
            h0 = scal(o_wr1 + 0) * m[0] + scal(o_wr1 + 1) * m[1] \
                + scal(o_wr1 + 2) * m[2] + scal(o_wr1 + 3) * m[3] \
                + scal(o_br1 + 0)
            h1 = scal(o_wr1 + 4) * m[0] + scal(o_wr1 + 5) * m[1] \
                + scal(o_wr1 + 6) * m[2] + scal(o_wr1 + 7) * m[3] \
                + scal(o_br1 + 1)
            g0 = scal(o_wr2 + 0) * h0 + scal(o_wr2 + 1) * h1 + scal(o_br2 + 0)
            g1 = scal(o_wr2 + 2) * h0 + scal(o_wr2 + 3) * h1 + scal(o_br2 + 1)
            return g0, g1

        gx0, gx1 = gates(mxs)
        gy0, gy1 = gates(mys)
        one = jnp.float32(1.0)
        zero = jnp.float32(0.0)
        mx0 = jnp.where(gx0 > 0, one, zero)
        mx1 = jnp.where(gx1 > 0, one, zero)
        my0 = jnp.where(gy0 > 0, one, zero)
        my1 = jnp.where(gy1 > 0, one, zero)

        cx0 = [[mx0 * scal(o_we1 + o * C + c) for c in range(C)]
               for o in range(co_x)]
        cy0 = [[my0 * scal(o_we3 + o * C + c) for c in range(C)]
               for o in range(co_x)]
        bb0 = [mx0 * scal(o_be1 + o) + my0 * scal(o_be3 + o)
               for o in range(co_x)]
        cx1 = [[mx1 * scal(o_we2 + o * C + c) for c in range(C)]
               for o in range(co_y)]
        cy1 = [[my1 * scal(o_we4 + o * C + c) for c in range(C)]
               for o in range(co_y)]
        bb1 = [mx1 * scal(o_be2 + o) + my1 * scal(o_be4 + o)
               for o in range(co_y)]

        def chunk(k, _):
            pltpu.async_copy(x_hbm.at[b, :, k, :], xbuf, sem).wait()
            pltpu.async_copy(y_hbm.at[b, :, k, :], ybuf, sem).wait()

            def mix(i, _2):
                sl = pl.ds(i * _L, _L)
                xs = [xbuf[c, sl] for c in range(C)]
                ys = [ybuf[c, sl] for c in range(C)]
                for o in range(co_x):
                    acc = cx0[o][0] * xs[0]
                    for c in range(1, C):
                        acc = acc + cx0[o][c] * xs[c]
                    for c in range(C):
                        acc = acc + cy0[o][c] * ys[c]
                    oxbuf[o, sl] = acc + bb0[o]
                for o in range(co_y):
                    acc = cx1[o][0] * xs[0]
                    for c in range(1, C):
                        acc = acc + cx1[o][c] * xs[c]
                    for c in range(C):
                        acc = acc + cy1[o][c] * ys[c]
                    oybuf[o, sl] = acc + bb1[o]
                return 0

            lax.fori_loop(0, nvec, mix, 0)

            pltpu.sync_copy(oxbuf, ox_hbm.at[b, :, k, :])
            pltpu.sync_copy(oybuf, oy_hbm.at[b, :, k, :])
            return 0

        lax.fori_loop(0, NCH, chunk, 0)


def kernel(x, y, W_r1, b_r1, W_r2, b_r2, W_e1, b_e1, W_e2, b_e2,
           W_e3, b_e3, W_e4, b_e4):
    B, C, H, W = x.shape
    P = H * W
    chw = P // _NCH
    co_x = W_e1.shape[0]
    co_y = W_e2.shape[0]

    x2 = x.reshape(B, C, _NCH, chw)
    y2 = y.reshape(B, C, _NCH, chw)

    params = jnp.concatenate([
        W_r1.reshape(-1), b_r1, W_r2.reshape(-1), b_r2,
        W_e1.reshape(-1), b_e1, W_e2.reshape(-1), b_e2,
        W_e3.reshape(-1), b_e3, W_e4.reshape(-1), b_e4,
        jnp.zeros(_NPV * _L - 16 - 2 * (co_x * C + co_x + co_y * C + co_y),
                  jnp.float32),
    ])

    mesh = plsc.VectorSubcoreMesh(core_axis_name="c", subcore_axis_name="s")
    fn = pl.kernel(
        functools.partial(_sc_body, co_x, co_y),
        mesh=mesh,
        out_type=[
            jax.ShapeDtypeStruct((B, co_x, _NCH, chw), jnp.float32),
            jax.ShapeDtypeStruct((B, co_y, _NCH, chw), jnp.float32),
        ],
        scratch_types=[
            pltpu.VMEM((C, chw), jnp.float32),
            pltpu.VMEM((C, chw), jnp.float32),
            pltpu.VMEM((co_x, chw), jnp.float32),
            pltpu.VMEM((co_y, chw), jnp.float32),
            pltpu.VMEM((_NPV * _L,), jnp.float32),
            pltpu.SemaphoreType.DMA,
        ],
    )
    out_x, out_y = fn(x2, y2, params)
    return out_x.reshape(B, co_x, H, W), out_y.reshape(B, co_y, H, W)


# speedup vs baseline: 14.7896x; 14.7896x over previous
import jax
import jax.numpy as jnp
from jax.experimental import pallas as pl
from jax.experimental.pallas import tpu as pltpu


def _body(wr1, br1, wr2, br2, we1, be1, we2, be2, we3, be3, we4, be4,
          xr, yr, ox, oy):
    for i in range(xr.shape[0]):
        _sample(wr1, br1, wr2, br2, we1, be1, we2, be2, we3, be3, we4, be4,
                xr, yr, ox, oy, i)


def _sample(wr1, br1, wr2, br2, we1, be1, we2, be2, we3, be3, we4, be4,
            xr, yr, ox, oy, i):
    x = [xr[i, c] for c in range(xr.shape[1])]
    y = [yr[i, c] for c in range(yr.shape[1])]
    inv = 1.0 / (x[0].shape[0] * x[0].shape[1])

    mx = [jnp.sum(v) * inv for v in x]
    my = [jnp.sum(v) * inv for v in y]

    def _gates(m):
        h0 = wr1[0, 0] * m[0] + wr1[0, 1] * m[1] + wr1[0, 2] * m[2] \
            + wr1[0, 3] * m[3] + br1[0]
        h1 = wr1[1, 0] * m[0] + wr1[1, 1] * m[1] + wr1[1, 2] * m[2] \
            + wr1[1, 3] * m[3] + br1[1]
        g0 = wr2[0, 0] * h0 + wr2[0, 1] * h1 + br2[0]
        g1 = wr2[1, 0] * h0 + wr2[1, 1] * h1 + br2[1]
        return g0, g1

    gx0, gx1 = _gates(mx)
    gy0, gy1 = _gates(my)
    one = jnp.float32(1.0)
    zero = jnp.float32(0.0)
    mx0 = jnp.where(gx0 > 0, one, zero)
    mx1 = jnp.where(gx1 > 0, one, zero)
    my0 = jnp.where(gy0 > 0, one, zero)
    my1 = jnp.where(gy1 > 0, one, zero)

    co_x, co_y = ox.shape[1], oy.shape[1]
    cx0 = [[mx0 * we1[o, c] for c in range(len(x))] for o in range(co_x)]
    cy0 = [[my0 * we3[o, c] for c in range(len(y))] for o in range(co_x)]
    b0 = [mx0 * be1[o] + my0 * be3[o] for o in range(co_x)]
    cx1 = [[mx1 * we2[o, c] for c in range(len(x))] for o in range(co_y)]
    cy1 = [[my1 * we4[o, c] for c in range(len(y))] for o in range(co_y)]
    b1 = [mx1 * be2[o] + my1 * be4[o] for o in range(co_y)]

    H = x[0].shape[0]
    CH = 16
    for k in range(0, H, CH):
        xc = [v[k:k + CH, :] for v in x]
        yc = [v[k:k + CH, :] for v in y]

        def _mix(cxs, cys, bias):
            acc = cxs[0] * xc[0]
            for c in range(1, len(xc)):
                acc = acc + cxs[c] * xc[c]
            for c in range(len(yc)):
                acc = acc + cys[c] * yc[c]
            return acc + bias

        for o in range(co_x):
            ox[i, o, k:k + CH, :] = _mix(cx0[o], cy0[o], b0[o])
        for o in range(co_y):
            oy[i, o, k:k + CH, :] = _mix(cx1[o], cy1[o], b1[o])


def kernel(x, y, W_r1, b_r1, W_r2, b_r2, W_e1, b_e1, W_e2, b_e2,
           W_e3, b_e3, W_e4, b_e4):
    B, C, H, W = x.shape
    co_x = W_e1.shape[0]
    co_y = W_e2.shape[0]

    smem = pl.BlockSpec(memory_space=pltpu.SMEM)

    NB = 2

    def big(c):
        return pl.BlockSpec((NB, c, H, W), lambda b: (b, 0, 0, 0))

    out_x, out_y = pl.pallas_call(
        _body,
        grid=(B // NB,),
        in_specs=[smem] * 12 + [big(C), big(C)],
        out_specs=[big(co_x), big(co_y)],
        out_shape=[
            jax.ShapeDtypeStruct((B, co_x, H, W), jnp.float32),
            jax.ShapeDtypeStruct((B, co_y, H, W), jnp.float32),
        ],
        compiler_params=pltpu.CompilerParams(
            dimension_semantics=("arbitrary",),
        ),
    )(W_r1, b_r1, W_r2, b_r2, W_e1, b_e1, W_e2, b_e2, W_e3, b_e3, W_e4,
      b_e4, x, y)

    return out_x, out_y


# speedup vs baseline: 15.9399x; 1.0778x over previous
import jax
import jax.numpy as jnp
from jax.experimental import pallas as pl
from jax.experimental.pallas import tpu as pltpu


def _body(wr1, br1, wr2, br2, we1, be1, we2, be2, we3, be3, we4, be4,
          xr, yr, ox, oy):
    for i in range(xr.shape[0]):
        _sample(wr1, br1, wr2, br2, we1, be1, we2, be2, we3, be3, we4, be4,
                xr, yr, ox, oy, i)


def _sample(wr1, br1, wr2, br2, we1, be1, we2, be2, we3, be3, we4, be4,
            xr, yr, ox, oy, i):
    x = [xr[i, c] for c in range(xr.shape[1])]
    y = [yr[i, c] for c in range(yr.shape[1])]
    inv = 1.0 / (x[0].shape[0] * x[0].shape[1])

    mx = [jnp.sum(v) * inv for v in x]
    my = [jnp.sum(v) * inv for v in y]

    def _gates(m):
        h0 = wr1[0, 0] * m[0] + wr1[0, 1] * m[1] + wr1[0, 2] * m[2] \
            + wr1[0, 3] * m[3] + br1[0]
        h1 = wr1[1, 0] * m[0] + wr1[1, 1] * m[1] + wr1[1, 2] * m[2] \
            + wr1[1, 3] * m[3] + br1[1]
        g0 = wr2[0, 0] * h0 + wr2[0, 1] * h1 + br2[0]
        g1 = wr2[1, 0] * h0 + wr2[1, 1] * h1 + br2[1]
        return g0, g1

    gx0, gx1 = _gates(mx)
    gy0, gy1 = _gates(my)
    one = jnp.float32(1.0)
    zero = jnp.float32(0.0)
    mx0 = jnp.where(gx0 > 0, one, zero)
    mx1 = jnp.where(gx1 > 0, one, zero)
    my0 = jnp.where(gy0 > 0, one, zero)
    my1 = jnp.where(gy1 > 0, one, zero)

    co_x, co_y = ox.shape[1], oy.shape[1]
    cx0 = [[mx0 * we1[o, c] for c in range(len(x))] for o in range(co_x)]
    cy0 = [[my0 * we3[o, c] for c in range(len(y))] for o in range(co_x)]
    b0 = [mx0 * be1[o] + my0 * be3[o] for o in range(co_x)]
    cx1 = [[mx1 * we2[o, c] for c in range(len(x))] for o in range(co_y)]
    cy1 = [[my1 * we4[o, c] for c in range(len(y))] for o in range(co_y)]
    b1 = [mx1 * be2[o] + my1 * be4[o] for o in range(co_y)]

    H = x[0].shape[0]
    CH = 16
    for k in range(0, H, CH):
        xc = [v[k:k + CH, :] for v in x]
        yc = [v[k:k + CH, :] for v in y]

        def _mix(cxs, cys, bias):
            acc = cxs[0] * xc[0]
            for c in range(1, len(xc)):
                acc = acc + cxs[c] * xc[c]
            for c in range(len(yc)):
                acc = acc + cys[c] * yc[c]
            return acc + bias

        for o in range(co_x):
            ox[i, o, k:k + CH, :] = _mix(cx0[o], cy0[o], b0[o])
        for o in range(co_y):
            oy[i, o, k:k + CH, :] = _mix(cx1[o], cy1[o], b1[o])


def kernel(x, y, W_r1, b_r1, W_r2, b_r2, W_e1, b_e1, W_e2, b_e2,
           W_e3, b_e3, W_e4, b_e4):
    B, C, H, W = x.shape
    co_x = W_e1.shape[0]
    co_y = W_e2.shape[0]

    smem = pl.BlockSpec(memory_space=pltpu.SMEM)

    NB = 4

    def big(c):
        return pl.BlockSpec((NB, c, H, W), lambda b: (b, 0, 0, 0))

    out_x, out_y = pl.pallas_call(
        _body,
        grid=(B // NB,),
        in_specs=[smem] * 12 + [big(C), big(C)],
        out_specs=[big(co_x), big(co_y)],
        out_shape=[
            jax.ShapeDtypeStruct((B, co_x, H, W), jnp.float32),
            jax.ShapeDtypeStruct((B, co_y, H, W), jnp.float32),
        ],
        compiler_params=pltpu.CompilerParams(
            dimension_semantics=("arbitrary",),
        ),
    )(W_r1, b_r1, W_r2, b_r2, W_e1, b_e1, W_e2, b_e2, W_e3, b_e3, W_e4,
      b_e4, x, y)

    return out_x, out_y


# speedup vs baseline: 15.9941x; 1.0034x over previous
import jax
import jax.numpy as jnp
from jax.experimental import pallas as pl
from jax.experimental.pallas import tpu as pltpu


def _body(wr1, br1, wr2, br2, we1, be1, we2, be2, we3, be3, we4, be4,
          xr, yr, ox, oy):
    for i in range(xr.shape[0]):
        _sample(wr1, br1, wr2, br2, we1, be1, we2, be2, we3, be3, we4, be4,
                xr, yr, ox, oy, i)


def _sample(wr1, br1, wr2, br2, we1, be1, we2, be2, we3, be3, we4, be4,
            xr, yr, ox, oy, i):
    x = [xr[i, c] for c in range(xr.shape[1])]
    y = [yr[i, c] for c in range(yr.shape[1])]
    inv = 1.0 / (x[0].shape[0] * x[0].shape[1])

    def _mean(v):
        acc = v[0:16, :]
        for k in range(16, v.shape[0], 16):
            acc = acc + v[k:k + 16, :]
        return jnp.sum(acc) * inv

    mx = [_mean(v) for v in x]
    my = [_mean(v) for v in y]

    def _gates(m):
        h0 = wr1[0, 0] * m[0] + wr1[0, 1] * m[1] + wr1[0, 2] * m[2] \
            + wr1[0, 3] * m[3] + br1[0]
        h1 = wr1[1, 0] * m[0] + wr1[1, 1] * m[1] + wr1[1, 2] * m[2] \
            + wr1[1, 3] * m[3] + br1[1]
        g0 = wr2[0, 0] * h0 + wr2[0, 1] * h1 + br2[0]
        g1 = wr2[1, 0] * h0 + wr2[1, 1] * h1 + br2[1]
        return g0, g1

    gx0, gx1 = _gates(mx)
    gy0, gy1 = _gates(my)
    one = jnp.float32(1.0)
    zero = jnp.float32(0.0)
    mx0 = jnp.where(gx0 > 0, one, zero)
    mx1 = jnp.where(gx1 > 0, one, zero)
    my0 = jnp.where(gy0 > 0, one, zero)
    my1 = jnp.where(gy1 > 0, one, zero)

    co_x, co_y = ox.shape[1], oy.shape[1]
    cx0 = [[mx0 * we1[o, c] for c in range(len(x))] for o in range(co_x)]
    cy0 = [[my0 * we3[o, c] for c in range(len(y))] for o in range(co_x)]
    b0 = [mx0 * be1[o] + my0 * be3[o] for o in range(co_x)]
    cx1 = [[mx1 * we2[o, c] for c in range(len(x))] for o in range(co_y)]
    cy1 = [[my1 * we4[o, c] for c in range(len(y))] for o in range(co_y)]
    b1 = [mx1 * be2[o] + my1 * be4[o] for o in range(co_y)]

    H = x[0].shape[0]
    CH = 16
    for k in range(0, H, CH):
        xc = [v[k:k + CH, :] for v in x]
        yc = [v[k:k + CH, :] for v in y]

        def _mix(cxs, cys, bias):
            acc = cxs[0] * xc[0]
            for c in range(1, len(xc)):
                acc = acc + cxs[c] * xc[c]
            for c in range(len(yc)):
                acc = acc + cys[c] * yc[c]
            return acc + bias

        for o in range(co_x):
            ox[i, o, k:k + CH, :] = _mix(cx0[o], cy0[o], b0[o])
        for o in range(co_y):
            oy[i, o, k:k + CH, :] = _mix(cx1[o], cy1[o], b1[o])


def kernel(x, y, W_r1, b_r1, W_r2, b_r2, W_e1, b_e1, W_e2, b_e2,
           W_e3, b_e3, W_e4, b_e4):
    B, C, H, W = x.shape
    co_x = W_e1.shape[0]
    co_y = W_e2.shape[0]

    smem = pl.BlockSpec(memory_space=pltpu.SMEM)

    NB = 4

    def big(c):
        return pl.BlockSpec((NB, c, H, W), lambda b: (b, 0, 0, 0))

    out_x, out_y = pl.pallas_call(
        _body,
        grid=(B // NB,),
        in_specs=[smem] * 12 + [big(C), big(C)],
        out_specs=[big(co_x), big(co_y)],
        out_shape=[
            jax.ShapeDtypeStruct((B, co_x, H, W), jnp.float32),
            jax.ShapeDtypeStruct((B, co_y, H, W), jnp.float32),
        ],
        compiler_params=pltpu.CompilerParams(
            dimension_semantics=("arbitrary",),
        ),
    )(W_r1, b_r1, W_r2, b_r2, W_e1, b_e1, W_e2, b_e2, W_e3, b_e3, W_e4,
      b_e4, x, y)

    return out_x, out_y
